# Initial kernel scaffold; baseline (speedup 1.0000x reference)
import jax
import jax.numpy as jnp
from jax.experimental import pallas as pl


def kernel(species_indices, moves_indices, abilities_indices, items_indices, species_unknown, species_data, species_W1, species_b1, species_W2, species_b2, species_W3, species_b3, moves_unknown, moves_data, moves_W1, moves_b1, moves_W2, moves_b2, moves_W3, moves_b3, abilities_unknown, abilities_data, abilities_W1, abilities_b1, abilities_W2, abilities_b2, abilities_W3, abilities_b3, items_unknown, items_data, items_W1, items_b1, items_W2, items_b2, items_W3, items_b3):
    raise NotImplementedError("write your pallas kernel here")


# speedup vs baseline: 4.1466x; 4.1466x over previous
import functools

import jax
import jax.numpy as jnp
from jax import lax
from jax.experimental import pallas as pl
from jax.experimental.pallas import tpu as pltpu
from jax.experimental.pallas import tpu_sc as plsc

B_TOTAL = 16384
D = 128

_SECTIONS = (
    ("species",   1025, 1,    0, 1032, True),
    ("moves",      920, 2, 1032,  928, False),
    ("abilities",  310, 1, 1960,  312, True),
    ("items",      560, 1, 2272,  568, True),
)
TABLE_ROWS = 2840


def _build_table_body(x_ref, *rest):
    (sw1, sb1, sw2, sb2, sw3, sb3,
     mw1, mb1, mw2, mb2, mw3, mb3,
     aw1, ab1, aw2, ab2, aw3, ab3,
     iw1, ib1, iw2, ib2, iw3, ib3, out_ref) = rest
    weights = {
        "species": (sw1, sb1, sw2, sb2, sw3, sb3),
        "moves": (mw1, mb1, mw2, mb2, mw3, mb3),
        "abilities": (aw1, ab1, aw2, ab2, aw3, ab3),
        "items": (iw1, ib1, iw2, ib2, iw3, ib3),
    }
    dn = (((1,), (1,)), ((), ()))
    for name, _n, nu, off, rows, norm in _SECTIONS:
        w1, b1, w2, b2, w3, b3 = weights[name]
        x = x_ref[off:off + rows, :]
        h = jax.nn.relu(
            lax.dot_general(x, w1[...], dn, precision=lax.Precision.HIGHEST,
                            preferred_element_type=jnp.float32) + b1[...])
        h = jax.nn.relu(
            lax.dot_general(h, w2[...], dn, precision=lax.Precision.HIGHEST,
                            preferred_element_type=jnp.float32) + b2[...])
        h = lax.dot_general(h, w3[...], dn, precision=lax.Precision.HIGHEST,
                            preferred_element_type=jnp.float32) + b3[...]
        rowid = lax.broadcasted_iota(jnp.int32, (rows, 1), 0)
        o = jnp.where(rowid < nu, x, h)
        if norm:
            n = jnp.sqrt(jnp.sum(o * o, axis=1, keepdims=True))
            o = o / jnp.maximum(n, 1e-12)
        out_ref[off:off + rows, :] = o


def _build_table(stacked, weight_list):
    return pl.pallas_call(
        _build_table_body,
        out_shape=jax.ShapeDtypeStruct((TABLE_ROWS, D), jnp.float32),
    )(stacked, *weight_list)


def _make_gather_kernel(chunk):
    mesh = plsc.VectorSubcoreMesh(core_axis_name="c", subcore_axis_name="s")
    nw = mesh.num_cores * mesh.num_subcores
    per_w = B_TOTAL // nw
    n_chunks = per_w // chunk
    vpt = (chunk * D) // 16

    def body(table, s_i, a_i, i_i, m0_i, m1_i, m2_i, m3_i, a_out, m_out,
             sidx, aidx, iidx, m0idx, m1idx, m2idx, m3idx,
             rs, ra, ri, rm0, rm1, rm2, rm3, sem):
        wid = lax.axis_index("s") * mesh.num_cores + lax.axis_index("c")

        def one_chunk(g, carry):
            base = wid * per_w + g * chunk
            pltpu.sync_copy(s_i.at[pl.ds(base, chunk)], sidx)
            pltpu.sync_copy(a_i.at[pl.ds(base, chunk)], aidx)
            pltpu.sync_copy(i_i.at[pl.ds(base, chunk)], iidx)
            pltpu.sync_copy(m0_i.at[pl.ds(base, chunk)], m0idx)
            pltpu.sync_copy(m1_i.at[pl.ds(base, chunk)], m1idx)
            pltpu.sync_copy(m2_i.at[pl.ds(base, chunk)], m2idx)
            pltpu.sync_copy(m3_i.at[pl.ds(base, chunk)], m3idx)
            c1 = pltpu.async_copy(table.at[sidx], rs, sem)
            c2 = pltpu.async_copy(table.at[aidx], ra, sem)
            c3 = pltpu.async_copy(table.at[iidx], ri, sem)
            c4 = pltpu.async_copy(table.at[m0idx], rm0, sem)
            c5 = pltpu.async_copy(table.at[m1idx], rm1, sem)
            c6 = pltpu.async_copy(table.at[m2idx], rm2, sem)
            c7 = pltpu.async_copy(table.at[m3idx], rm3, sem)
            c1.wait(); c2.wait(); c3.wait(); c4.wait()
            c5.wait(); c6.wait(); c7.wait()

            def accum(i, c):
                r = i // (D // 16)
                k = (i % (D // 16)) * 16
                sl = pl.ds(k, 16)
                rs[r, sl] = rs[r, sl] + ra[r, sl] + ri[r, sl]
                rm0[r, sl] = (rm0[r, sl] + rm1[r, sl]) + (rm2[r, sl] + rm3[r, sl])
                return c
            lax.fori_loop(0, vpt, accum, 0, unroll=8)
            pltpu.sync_copy(rs, a_out.at[pl.ds(base, chunk)])
            pltpu.sync_copy(rm0, m_out.at[pl.ds(base, chunk)])
            return carry

        lax.fori_loop(0, n_chunks, one_chunk, 0)

    idx_t = jax.ShapeDtypeStruct((chunk,), jnp.int32)
    row_t = jax.ShapeDtypeStruct((chunk, D), jnp.float32)
    return pl.kernel(
        body,
        out_type=(jax.ShapeDtypeStruct((B_TOTAL, D), jnp.float32),
                  jax.ShapeDtypeStruct((B_TOTAL, D), jnp.float32)),
        mesh=mesh,
        scratch_types=(
            pltpu.VMEM(idx_t.shape, idx_t.dtype),
            pltpu.VMEM(idx_t.shape, idx_t.dtype),
            pltpu.VMEM(idx_t.shape, idx_t.dtype),
            pltpu.VMEM(idx_t.shape, idx_t.dtype),
            pltpu.VMEM(idx_t.shape, idx_t.dtype),
            pltpu.VMEM(idx_t.shape, idx_t.dtype),
            pltpu.VMEM(idx_t.shape, idx_t.dtype),
            pltpu.VMEM(row_t.shape, row_t.dtype),
            pltpu.VMEM(row_t.shape, row_t.dtype),
            pltpu.VMEM(row_t.shape, row_t.dtype),
            pltpu.VMEM(row_t.shape, row_t.dtype),
            pltpu.VMEM(row_t.shape, row_t.dtype),
            pltpu.VMEM(row_t.shape, row_t.dtype),
            pltpu.VMEM(row_t.shape, row_t.dtype),
            pltpu.SemaphoreType.DMA,
        ),
    )


def _epilogue_body(a_ref, m_ref, out_ref):
    m = m_ref[...]
    n = jnp.sqrt(jnp.sum(m * m, axis=1, keepdims=True))
    out_ref[...] = a_ref[...] + m / jnp.maximum(n, 1e-12)


def _epilogue(a, m):
    blk = 2048
    return pl.pallas_call(
        _epilogue_body,
        grid=(B_TOTAL // blk,),
        in_specs=[pl.BlockSpec((blk, D), lambda i: (i, 0)),
                  pl.BlockSpec((blk, D), lambda i: (i, 0))],
        out_specs=pl.BlockSpec((blk, D), lambda i: (i, 0)),
        out_shape=jax.ShapeDtypeStruct((B_TOTAL, D), jnp.float32),
    )(a, m)


def kernel(species_indices, moves_indices, abilities_indices, items_indices, species_unknown, species_data, species_W1, species_b1, species_W2, species_b2, species_W3, species_b3, moves_unknown, moves_data, moves_W1, moves_b1, moves_W2, moves_b2, moves_W3, moves_b3, abilities_unknown, abilities_data, abilities_W1, abilities_b1, abilities_W2, abilities_b2, abilities_W3, abilities_b3, items_unknown, items_data, items_W1, items_b1, items_W2, items_b2, items_W3, items_b3):
    f32 = jnp.float32
    parts = []
    for (name, n, nu, off, rows, _norm), unk, data in zip(
            _SECTIONS,
            (species_unknown, moves_unknown, abilities_unknown, items_unknown),
            (species_data, moves_data, abilities_data, items_data)):
        pad = rows - nu - n
        parts.append(unk.astype(f32))
        parts.append(data.astype(f32))
        if pad:
            parts.append(jnp.zeros((pad, D), f32))
    stacked = jnp.concatenate(parts, axis=0)

    weight_list = [w.astype(f32) for w in (
        species_W1, species_b1, species_W2, species_b2, species_W3, species_b3,
        moves_W1, moves_b1, moves_W2, moves_b2, moves_W3, moves_b3,
        abilities_W1, abilities_b1, abilities_W2, abilities_b2, abilities_W3, abilities_b3,
        items_W1, items_b1, items_W2, items_b2, items_W3, items_b3)]
    table = _build_table(stacked, weight_list)

    i32 = jnp.int32
    s_i = (jnp.clip(species_indices, 0, 1025) + 0).astype(i32)
    a_i = (jnp.clip(abilities_indices, 0, 310) + 1960).astype(i32)
    i_i = (jnp.clip(items_indices, 0, 560) + 2272).astype(i32)
    m = jnp.clip(moves_indices, 0, 921).astype(i32) + 1032
    m0_i, m1_i, m2_i, m3_i = m[:, 0], m[:, 1], m[:, 2], m[:, 3]

    gather = _make_gather_kernel(chunk=64)
    a_sum, m_sum = gather(table, s_i, a_i, i_i, m0_i, m1_i, m2_i, m3_i)
    return _epilogue(a_sum, m_sum)


# speedup vs baseline: 5.3679x; 1.2945x over previous
import functools

import jax
import jax.numpy as jnp
from jax import lax
from jax.experimental import pallas as pl
from jax.experimental.pallas import tpu as pltpu
from jax.experimental.pallas import tpu_sc as plsc

B_TOTAL = 16384
D = 128

_SECTIONS = (
    ("species",   1025, 1,    0, 1032, True),
    ("moves",      920, 2, 1032,  928, False),
    ("abilities",  310, 1, 1960,  312, True),
    ("items",      560, 1, 2272,  568, True),
)
TABLE_ROWS = 2840


def _build_table_body(x_ref, *rest):
    (sw1, sb1, sw2, sb2, sw3, sb3,
     mw1, mb1, mw2, mb2, mw3, mb3,
     aw1, ab1, aw2, ab2, aw3, ab3,
     iw1, ib1, iw2, ib2, iw3, ib3, out_ref) = rest
    weights = {
        "species": (sw1, sb1, sw2, sb2, sw3, sb3),
        "moves": (mw1, mb1, mw2, mb2, mw3, mb3),
        "abilities": (aw1, ab1, aw2, ab2, aw3, ab3),
        "items": (iw1, ib1, iw2, ib2, iw3, ib3),
    }
    dn = (((1,), (1,)), ((), ()))
    for name, _n, nu, off, rows, norm in _SECTIONS:
        w1, b1, w2, b2, w3, b3 = weights[name]
        x = x_ref[off:off + rows, :]
        h = jax.nn.relu(
            lax.dot_general(x, w1[...], dn, precision=lax.Precision.HIGHEST,
                            preferred_element_type=jnp.float32) + b1[...])
        h = jax.nn.relu(
            lax.dot_general(h, w2[...], dn, precision=lax.Precision.HIGHEST,
                            preferred_element_type=jnp.float32) + b2[...])
        h = lax.dot_general(h, w3[...], dn, precision=lax.Precision.HIGHEST,
                            preferred_element_type=jnp.float32) + b3[...]
        rowid = lax.broadcasted_iota(jnp.int32, (rows, 1), 0)
        o = jnp.where(rowid < nu, x, h)
        if norm:
            n = jnp.sqrt(jnp.sum(o * o, axis=1, keepdims=True))
            o = o / jnp.maximum(n, 1e-12)
        out_ref[off:off + rows, :] = o


def _build_table(stacked, weight_list):
    return pl.pallas_call(
        _build_table_body,
        out_shape=jax.ShapeDtypeStruct((TABLE_ROWS, D), jnp.float32),
    )(stacked, *weight_list)


_CHUNK = 32
_RPT = 7
_HALF = _RPT * _CHUNK // 2


def _make_gather_kernel():
    mesh = plsc.VectorSubcoreMesh(core_axis_name="c", subcore_axis_name="s")
    nw = mesh.num_cores * mesh.num_subcores
    per_w = B_TOTAL // nw
    n_chunks = per_w // _CHUNK

    def body(table, idx_hbm, a_out, m_out,
             idxv, rows0, rows1, ab0, mb0, ab1, mb1,
             gsem0, gsem1, ssem0, ssem1):
        wid = lax.axis_index("s") * mesh.num_cores + lax.axis_index("c")
        pltpu.sync_copy(idx_hbm.at[wid], idxv)
        rows = (rows0, rows1)
        ab = (ab0, ab1)
        mb = (mb0, mb1)
        gsem = (gsem0, gsem1)
        ssem = (ssem0, ssem1)

        def fire_gather(g, buf):
            s = buf & 1
            return (pltpu.async_copy(table.at[idxv.at[2 * g]],
                                     rows[s].at[0:_HALF], gsem[s]),
                    pltpu.async_copy(table.at[idxv.at[2 * g + 1]],
                                     rows[s].at[_HALF:2 * _HALF], gsem[s]))

        gd = [None, None]
        sd = [None, None]
        gd[0] = fire_gather(0, 0)
        for g in range(n_chunks):
            cur = g & 1
            if g + 1 < n_chunks:
                gd[1 - cur] = fire_gather(g + 1, 1 - cur)
            gd[cur][0].wait()
            gd[cur][1].wait()
            if sd[cur] is not None:
                sd[cur][0].wait()
                sd[cur][1].wait()
            rbuf = rows[cur]
            abuf = ab[cur]
            mbuf = mb[cur]

            def accum(t, c):
                r7 = t * _RPT
                for k in range(D // 16):
                    sl = pl.ds(k * 16, 16)
                    abuf[t, sl] = (rbuf[r7, sl] + rbuf[r7 + 5, sl]) + rbuf[r7 + 6, sl]
                    mbuf[t, sl] = ((rbuf[r7 + 1, sl] + rbuf[r7 + 2, sl])
                                   + (rbuf[r7 + 3, sl] + rbuf[r7 + 4, sl]))
                return c
            lax.fori_loop(0, _CHUNK, accum, 0)
            base = wid * per_w + g * _CHUNK
            sd[cur] = (pltpu.async_copy(abuf, a_out.at[pl.ds(base, _CHUNK)], ssem[cur]),
                       pltpu.async_copy(mbuf, m_out.at[pl.ds(base, _CHUNK)], ssem[cur]))
        for s in (0, 1):
            if sd[s] is not None:
                sd[s][0].wait()
                sd[s][1].wait()

    return nw, n_chunks, pl.kernel(
        body,
        out_type=(jax.ShapeDtypeStruct((B_TOTAL, D), jnp.float32),
                  jax.ShapeDtypeStruct((B_TOTAL, D), jnp.float32)),
        mesh=mesh,
        scratch_types=(
            pltpu.VMEM((2 * n_chunks, _HALF), jnp.int32),
            pltpu.VMEM((2 * _HALF, D), jnp.float32),
            pltpu.VMEM((2 * _HALF, D), jnp.float32),
            pltpu.VMEM((_CHUNK, D), jnp.float32),
            pltpu.VMEM((_CHUNK, D), jnp.float32),
            pltpu.VMEM((_CHUNK, D), jnp.float32),
            pltpu.VMEM((_CHUNK, D), jnp.float32),
            pltpu.SemaphoreType.DMA,
            pltpu.SemaphoreType.DMA,
            pltpu.SemaphoreType.DMA,
            pltpu.SemaphoreType.DMA,
        ),
    )


def _epilogue_body(a_ref, m_ref, out_ref):
    m = m_ref[...]
    n = jnp.sqrt(jnp.sum(m * m, axis=1, keepdims=True))
    out_ref[...] = a_ref[...] + m / jnp.maximum(n, 1e-12)


def _epilogue(a, m):
    blk = 2048
    return pl.pallas_call(
        _epilogue_body,
        grid=(B_TOTAL // blk,),
        in_specs=[pl.BlockSpec((blk, D), lambda i: (i, 0)),
                  pl.BlockSpec((blk, D), lambda i: (i, 0))],
        out_specs=pl.BlockSpec((blk, D), lambda i: (i, 0)),
        out_shape=jax.ShapeDtypeStruct((B_TOTAL, D), jnp.float32),
    )(a, m)


def kernel(species_indices, moves_indices, abilities_indices, items_indices, species_unknown, species_data, species_W1, species_b1, species_W2, species_b2, species_W3, species_b3, moves_unknown, moves_data, moves_W1, moves_b1, moves_W2, moves_b2, moves_W3, moves_b3, abilities_unknown, abilities_data, abilities_W1, abilities_b1, abilities_W2, abilities_b2, abilities_W3, abilities_b3, items_unknown, items_data, items_W1, items_b1, items_W2, items_b2, items_W3, items_b3):
    f32 = jnp.float32
    parts = []
    for (name, n, nu, off, rows, _norm), unk, data in zip(
            _SECTIONS,
            (species_unknown, moves_unknown, abilities_unknown, items_unknown),
            (species_data, moves_data, abilities_data, items_data)):
        pad = rows - nu - n
        parts.append(unk.astype(f32))
        parts.append(data.astype(f32))
        if pad:
            parts.append(jnp.zeros((pad, D), f32))
    stacked = jnp.concatenate(parts, axis=0)

    weight_list = [w.astype(f32) for w in (
        species_W1, species_b1, species_W2, species_b2, species_W3, species_b3,
        moves_W1, moves_b1, moves_W2, moves_b2, moves_W3, moves_b3,
        abilities_W1, abilities_b1, abilities_W2, abilities_b2, abilities_W3, abilities_b3,
        items_W1, items_b1, items_W2, items_b2, items_W3, items_b3)]
    table = _build_table(stacked, weight_list)

    i32 = jnp.int32
    s_i = (jnp.clip(species_indices, 0, 1025) + 0).astype(i32)
    a_i = (jnp.clip(abilities_indices, 0, 310) + 1960).astype(i32)
    i_i = (jnp.clip(items_indices, 0, 560) + 2272).astype(i32)
    m = jnp.clip(moves_indices, 0, 921).astype(i32) + 1032

    nw, n_chunks, gather = _make_gather_kernel()
    idx_all = jnp.concatenate(
        [s_i[:, None], m, a_i[:, None], i_i[:, None]], axis=1)
    idx_hbm = idx_all.reshape(nw, 2 * n_chunks, _HALF)
    a_sum, m_sum = gather(table, idx_hbm)
    return _epilogue(a_sum, m_sum)
